# Initial kernel scaffold; baseline (speedup 1.0000x reference)
import jax
import jax.numpy as jnp
from jax.experimental import pallas as pl


def kernel(x, edge_index, W1, a_src1, a_dst1, b1, g1, be1, W2, a_src2, a_dst2, b2, Wf, bf):
    raise NotImplementedError("write your pallas kernel here")


# speedup vs baseline: 1.3717x; 1.3717x over previous
import functools
import jax
import jax.numpy as jnp
from jax.experimental import pallas as pl
from jax.experimental.pallas import tpu as pltpu

N = 10000
E = 160000
D = 256
C = 256
OUT = 128

_BM = 1000


def _mm_body(a_ref, b_ref, o_ref):
    o_ref[...] = jnp.dot(a_ref[...], b_ref[...], preferred_element_type=jnp.float32)


def _mm(a, b):
    m, k = a.shape
    n = b.shape[1]
    grid = (m // _BM,)
    return pl.pallas_call(
        _mm_body,
        grid=grid,
        in_specs=[
            pl.BlockSpec((_BM, k), lambda i: (i, 0)),
            pl.BlockSpec((k, n), lambda i: (0, 0)),
        ],
        out_specs=pl.BlockSpec((_BM, n), lambda i: (i, 0)),
        out_shape=jax.ShapeDtypeStruct((m, n), jnp.float32),
    )(a, b)


def _edge_phase(h, asv, adv, src, dst):
    alpha = asv[src] + adv[dst]
    alpha = jnp.maximum(alpha, 0.2 * alpha)
    w = jnp.exp(alpha)
    denom = jax.ops.segment_sum(w, dst, num_segments=N)
    seg = jax.ops.segment_sum(h[src] * w[:, None], dst, num_segments=N)
    return seg / (denom[:, None] + 1e-16)


def kernel(x, edge_index, W1, a_src1, a_dst1, b1, g1, be1, W2, a_src2, a_dst2, b2, Wf, bf):
    loop = jnp.arange(N, dtype=edge_index.dtype)
    src = jnp.concatenate([edge_index[0], loop])
    dst = jnp.concatenate([edge_index[1], loop])

    h1p = _mm(x, W1)
    as1 = h1p @ a_src1[0]
    ad1 = h1p @ a_dst1[0]
    u = _edge_phase(h1p, as1, ad1, src, dst) + b1
    m = u.mean(axis=0)
    v = u.var(axis=0)
    h1 = (u - m) / jnp.sqrt(v + 1e-5) * g1 + be1
    h1 = jax.nn.elu(h1)

    h2p = _mm(h1, W2)
    as2 = h2p @ a_src2[0]
    ad2 = h2p @ a_dst2[0]
    h2 = _edge_phase(h2p, as2, ad2, src, dst) + b2

    jk = jnp.maximum(h1, h2)
    return _mm(jk, Wf) + bf


# speedup vs baseline: 7.5360x; 5.4937x over previous
import jax
import jax.numpy as jnp
from jax import lax
from jax.experimental import pallas as pl
from jax.experimental.pallas import tpu as pltpu
from jax.experimental.pallas import tpu_sc as plsc

N = 10000
E = 160000
D = 256
C = 256
OUT = 128

NC = 2
NS = 16
LN = 16

ROWS_PER_TILE = 624
E_TOT = E + N
CHUNK = 64
EPT = 10688
E_PAD = NS * EPT
N_CHUNKS = EPT // CHUNK
DR = 79

_BM = 1000


def _mm_body(a_ref, b_ref, o_ref):
    o_ref[...] = jnp.dot(a_ref[...], b_ref[...], preferred_element_type=jnp.float32)


def _mm(a, b):
    m, k = a.shape
    n = b.shape[1]
    return pl.pallas_call(
        _mm_body,
        grid=(m // _BM,),
        in_specs=[
            pl.BlockSpec((_BM, k), lambda i: (i, 0)),
            pl.BlockSpec((k, n), lambda i: (0, 0)),
        ],
        out_specs=pl.BlockSpec((_BM, n), lambda i: (i, 0)),
        out_shape=jax.ShapeDtypeStruct((m, n), jnp.float32),
    )(a, b)


def _rot(x, sh):
    idx = (lax.iota(jnp.int32, LN) + sh) % LN
    dnums = lax.GatherDimensionNumbers(
        offset_dims=(), collapsed_slice_dims=(0,), start_index_map=(0,))
    return lax.gather(x, idx[:, None], dnums, (1,),
                      mode=lax.GatherScatterMode.PROMISE_IN_BOUNDS)


def _edge_sc_body(src_hbm, dst_hbm, asv_hbm, adv_hbm, hl_hbm, hr_hbm,
                  sl_hbm, sr_hbm, den_hbm,
                  asv_v, adv_v, w_v, srcbuf, dstbuf, rowbuf, denom_v, idxr,
                  acc_sh, dsh_sh, sem):
    c = lax.axis_index("c")
    s = lax.axis_index("s")
    ebase = s * EPT
    rbase = s * ROWS_PER_TILE
    iota16 = lax.iota(jnp.int32, LN)
    zf16 = jnp.zeros((LN,), jnp.float32)

    pltpu.sync_copy(asv_hbm, asv_v)
    pltpu.sync_copy(adv_hbm, adv_v)

    def _zero2d(ref, rows):
        def body(i, _):
            def inner(j, _):
                ref[i, pl.ds(j * LN, LN)] = zf16
                return 0
            lax.fori_loop(0, 128 // LN, inner, 0)
            return 0
        lax.fori_loop(0, rows, body, 0)

    _zero2d(rowbuf, CHUNK)
    _zero2d(denom_v, DR)
    for j in range(DR // LN + 1):
        base = min(j * LN, DR - LN)
        idxr[pl.ds(base, LN)] = base + iota16

    @pl.when(s == 0)
    def _():
        pltpu.sync_copy(denom_v, dsh_sh)
    for off in range(0, ROWS_PER_TILE, CHUNK):
        nrows = min(CHUNK, ROWS_PER_TILE - off)
        pltpu.sync_copy(rowbuf.at[pl.ds(0, nrows)],
                        acc_sh.at[pl.ds(rbase + off, nrows)])

    @pl.when(s == NS - 1)
    def _():
        pltpu.sync_copy(rowbuf.at[pl.ds(0, N - NS * ROWS_PER_TILE)],
                        acc_sh.at[pl.ds(NS * ROWS_PER_TILE, N - NS * ROWS_PER_TILE)])

    def _p1(g, _):
        pltpu.sync_copy(src_hbm.at[pl.ds(ebase + g * CHUNK, CHUNK)], srcbuf)
        pltpu.sync_copy(dst_hbm.at[pl.ds(ebase + g * CHUNK, CHUNK)], dstbuf)

        def grp(j, _):
            s16 = srcbuf[pl.ds(j * LN, LN)]
            d16 = dstbuf[pl.ds(j * LN, LN)]
            a = plsc.load_gather(asv_v, [s16]) + plsc.load_gather(adv_v, [d16])
            a = jnp.maximum(a, 0.2 * a)
            w = jnp.exp(a)
            eid = ebase + g * CHUNK + j * LN + iota16
            w = jnp.where(eid < E_TOT, w, 0.0)
            w_v[pl.ds(g * CHUNK + j * LN, LN)] = w

            ds_, ws = plsc.sort_key_val(d16, w)
            csum = plsc.cumsum(ws)
            d_next = _rot(ds_, 1)
            d_prev = _rot(ds_, -1)
            c_prev = _rot(csum, -1)
            is_last = (ds_ != d_next) | (iota16 == LN - 1)
            is_start = (ds_ != d_prev) & (iota16 > 0)
            row16 = lax.shift_right_logical(ds_, 7)
            col16 = lax.bitwise_and(ds_, 127)
            plsc.addupdate_scatter(denom_v, [row16, col16], csum, mask=is_last)
            plsc.addupdate_scatter(denom_v, [row16, col16], -c_prev, mask=is_start)
            return 0

        lax.fori_loop(0, CHUNK // LN, grp, 0)
        return 0

    lax.fori_loop(0, N_CHUNKS, _p1, 0)

    plsc.subcore_barrier()

    def _chunk_body(hq_hbm):
        def body(g, _):
            pltpu.sync_copy(src_hbm.at[pl.ds(ebase + g * CHUNK, CHUNK)], srcbuf)
            pltpu.sync_copy(dst_hbm.at[pl.ds(ebase + g * CHUNK, CHUNK)], dstbuf)
            pltpu.async_copy(hq_hbm.at[srcbuf], rowbuf, sem).wait()

            def _scale(e, _):
                widx = jnp.full((LN,), g * CHUNK + e, jnp.int32)
                wspl = plsc.load_gather(w_v, [widx])
                for cg in range(128 // LN):
                    rowbuf[e, pl.ds(cg * LN, LN)] = rowbuf[e, pl.ds(cg * LN, LN)] * wspl
                return 0
            lax.fori_loop(0, CHUNK, _scale, 0)

            pltpu.sync_copy(rowbuf, acc_sh.at[dstbuf], add=True)
            return 0
        lax.fori_loop(0, N_CHUNKS, body, 0)

    @pl.when(c == 0)
    def _():
        _chunk_body(hl_hbm)

    @pl.when(c != 0)
    def _():
        _chunk_body(hr_hbm)

    pltpu.sync_copy(denom_v, dsh_sh.at[idxr], add=True)

    plsc.subcore_barrier()

    tail = N - NS * ROWS_PER_TILE

    @pl.when(c == 0)
    def _():
        pltpu.sync_copy(acc_sh.at[pl.ds(rbase, ROWS_PER_TILE)],
                        sl_hbm.at[pl.ds(rbase, ROWS_PER_TILE)])

        @pl.when(s == NS - 1)
        def _():
            pltpu.sync_copy(acc_sh.at[pl.ds(NS * ROWS_PER_TILE, tail)],
                            sl_hbm.at[pl.ds(NS * ROWS_PER_TILE, tail)])

        @pl.when(s == 0)
        def _():
            pltpu.sync_copy(dsh_sh, den_hbm)

    @pl.when(c != 0)
    def _():
        pltpu.sync_copy(acc_sh.at[pl.ds(rbase, ROWS_PER_TILE)],
                        sr_hbm.at[pl.ds(rbase, ROWS_PER_TILE)])

        @pl.when(s == NS - 1)
        def _():
            pltpu.sync_copy(acc_sh.at[pl.ds(NS * ROWS_PER_TILE, tail)],
                            sr_hbm.at[pl.ds(NS * ROWS_PER_TILE, tail)])


def _edge_sc(src, dst, asv, adv, hl, hr):
    mesh = plsc.VectorSubcoreMesh(core_axis_name="c", subcore_axis_name="s",
                                  num_cores=NC, num_subcores=NS)
    f = pl.kernel(
        _edge_sc_body,
        out_type=(
            jax.ShapeDtypeStruct((N, 128), jnp.float32),
            jax.ShapeDtypeStruct((N, 128), jnp.float32),
            jax.ShapeDtypeStruct((DR, 128), jnp.float32),
        ),
        mesh=mesh,
        scratch_types=[
            pltpu.VMEM((N, ), jnp.float32),
            pltpu.VMEM((N, ), jnp.float32),
            pltpu.VMEM((EPT,), jnp.float32),
            pltpu.VMEM((CHUNK,), jnp.int32),
            pltpu.VMEM((CHUNK,), jnp.int32),
            pltpu.VMEM((CHUNK, 128), jnp.float32),
            pltpu.VMEM((DR, 128), jnp.float32),
            pltpu.VMEM((DR,), jnp.int32),
            pltpu.VMEM_SHARED((N, 128), jnp.float32),
            pltpu.VMEM_SHARED((DR, 128), jnp.float32),
            pltpu.SemaphoreType.DMA,
        ],
        compiler_params=pltpu.CompilerParams(needs_layout_passes=False),
    )
    return f(src, dst, asv, adv, hl, hr)


def _edge_phase(h, asv, adv, src_pad, dst_pad):
    sl, sr, den = _edge_sc(src_pad, dst_pad, asv, adv, h[:, :128], h[:, 128:])
    seg = jnp.concatenate([sl, sr], axis=1)
    return seg / den.reshape(-1)[:N, None]


def kernel(x, edge_index, W1, a_src1, a_dst1, b1, g1, be1, W2, a_src2, a_dst2, b2, Wf, bf):
    loop = jnp.arange(N, dtype=edge_index.dtype)
    pad = jnp.zeros((E_PAD - E_TOT,), edge_index.dtype)
    src_pad = jnp.concatenate([edge_index[0], loop, pad])
    dst_pad = jnp.concatenate([edge_index[1], loop, pad])

    h1p = _mm(x, W1)
    as1 = h1p @ a_src1[0]
    ad1 = h1p @ a_dst1[0]
    u = _edge_phase(h1p, as1, ad1, src_pad, dst_pad) + b1
    m = u.mean(axis=0)
    v = u.var(axis=0)
    h1 = (u - m) / jnp.sqrt(v + 1e-5) * g1 + be1
    h1 = jax.nn.elu(h1)

    h2p = _mm(h1, W2)
    as2 = h2p @ a_src2[0]
    ad2 = h2p @ a_dst2[0]
    h2 = _edge_phase(h2p, as2, ad2, src_pad, dst_pad) + b2

    jk = jnp.maximum(h1, h2)
    return _mm(jk, Wf) + bf


# speedup vs baseline: 13.0962x; 1.7378x over previous
import jax
import jax.numpy as jnp
from jax import lax
from jax.experimental import pallas as pl
from jax.experimental.pallas import tpu as pltpu
from jax.experimental.pallas import tpu_sc as plsc

N = 10000
E = 160000
D = 256
C = 256
OUT = 128

NC = 2
NS = 16
LN = 16

ROWS_PER_TILE = 624
E_TOT = E + N
CHUNK = 64
BLK = 4
BLKE = BLK * CHUNK
N_CHUNKS = 168
N_BLKS = N_CHUNKS // BLK
EPT = N_CHUNKS * CHUNK
E_PAD = NS * EPT
E_IDX = E_PAD + BLKE
DR = 79

_BM = 1000


def _mm_body(a_ref, b_ref, o_ref):
    o_ref[...] = jnp.dot(a_ref[...], b_ref[...], preferred_element_type=jnp.float32)


def _mm(a, b):
    m, k = a.shape
    n = b.shape[1]
    return pl.pallas_call(
        _mm_body,
        grid=(m // _BM,),
        in_specs=[
            pl.BlockSpec((_BM, k), lambda i: (i, 0)),
            pl.BlockSpec((k, n), lambda i: (0, 0)),
        ],
        out_specs=pl.BlockSpec((_BM, n), lambda i: (i, 0)),
        out_shape=jax.ShapeDtypeStruct((m, n), jnp.float32),
    )(a, b)


def _rot(x, sh):
    idx = (lax.iota(jnp.int32, LN) + sh) % LN
    dnums = lax.GatherDimensionNumbers(
        offset_dims=(), collapsed_slice_dims=(0,), start_index_map=(0,))
    return lax.gather(x, idx[:, None], dnums, (1,),
                      mode=lax.GatherScatterMode.PROMISE_IN_BOUNDS)


def _edge_sc_body(src_hbm, dst_hbm, asv_hbm, adv_hbm, hl_hbm, hr_hbm,
                  sl_hbm, sr_hbm, den_hbm,
                  asv_v, adv_v,
                  srcblk0, srcblk1, dstblk0, dstblk1,
                  wch0, wch1, dstb0, dstb1, row0, row1,
                  denom_v, idxr, acc_sh, dsh_sh,
                  sem_b0, sem_b1, sem_g0, sem_g1, sem_s0, sem_s1):
    srcblk = [srcblk0, srcblk1]
    dstblk = [dstblk0, dstblk1]
    wch = [wch0, wch1]
    dstb = [dstb0, dstb1]
    row = [row0, row1]
    sem_b = [sem_b0, sem_b1]
    sem_g = [sem_g0, sem_g1]
    sem_s = [sem_s0, sem_s1]

    c = lax.axis_index("c")
    s = lax.axis_index("s")
    ebase = s * EPT
    rbase = s * ROWS_PER_TILE
    iota16 = lax.iota(jnp.int32, LN)
    zf16 = jnp.zeros((LN,), jnp.float32)

    pltpu.sync_copy(asv_hbm, asv_v)
    pltpu.sync_copy(adv_hbm, adv_v)

    def _zero2d(ref, rows):
        def body(i, _):
            def inner(j, _):
                ref[i, pl.ds(j * LN, LN)] = zf16
                return 0
            lax.fori_loop(0, 128 // LN, inner, 0)
            return 0
        lax.fori_loop(0, rows, body, 0)

    _zero2d(row0, CHUNK)
    _zero2d(denom_v, DR)
    for j in range(DR // LN + 1):
        base = min(j * LN, DR - LN)
        idxr[pl.ds(base, LN)] = base + iota16

    @pl.when(s == 0)
    def _():
        pltpu.sync_copy(denom_v, dsh_sh)
    for off in range(0, ROWS_PER_TILE, CHUNK):
        nrows = min(CHUNK, ROWS_PER_TILE - off)
        pltpu.sync_copy(row0.at[pl.ds(0, nrows)],
                        acc_sh.at[pl.ds(rbase + off, nrows)])

    tail = N - NS * ROWS_PER_TILE

    @pl.when(s == NS - 1)
    def _():
        pltpu.sync_copy(row0.at[pl.ds(0, tail)],
                        acc_sh.at[pl.ds(NS * ROWS_PER_TILE, tail)])

    def _wcompute(blk, kk, b, b2):
        for j in range(CHUNK // LN):
            s16 = srcblk[b][pl.ds(kk * CHUNK + j * LN, LN)]
            d16 = dstblk[b][pl.ds(kk * CHUNK + j * LN, LN)]
            dstb[b2][pl.ds(j * LN, LN)] = d16
            a = plsc.load_gather(asv_v, [s16]) + plsc.load_gather(adv_v, [d16])
            a = jnp.maximum(a, 0.2 * a)
            w = jnp.exp(a)
            eid = ebase + (blk * BLK + kk) * CHUNK + j * LN + iota16
            w = jnp.where(eid < E_TOT, w, 0.0)
            wch[b2][pl.ds(j * LN, LN)] = w

            ds_, ws = plsc.sort_key_val(d16, w)
            csum = plsc.cumsum(ws)
            d_next = _rot(ds_, 1)
            d_prev = _rot(ds_, -1)
            c_prev = _rot(csum, -1)
            is_last = (ds_ != d_next) | (iota16 == LN - 1)
            is_start = (ds_ != d_prev) & (iota16 > 0)
            row16 = lax.shift_right_logical(ds_, 7)
            col16 = lax.bitwise_and(ds_, 127)
            plsc.addupdate_scatter(denom_v, [row16, col16], csum, mask=is_last)
            plsc.addupdate_scatter(denom_v, [row16, col16], -c_prev, mask=is_start)

    def _scale(b2):
        def body(e, _):
            widx = jnp.full((LN,), e, jnp.int32)
            wspl = plsc.load_gather(wch[b2], [widx])
            for cg in range(128 // LN):
                row[b2][e, pl.ds(cg * LN, LN)] = row[b2][e, pl.ds(cg * LN, LN)] * wspl
            return 0
        lax.fori_loop(0, CHUNK, body, 0)

    def _block_body(h_hbm, blk, nb_):
        b = 1 - nb_
        off = ebase + (blk + 1) * BLKE
        fs = pltpu.async_copy(src_hbm.at[pl.ds(off, BLKE)], srcblk[nb_], sem_b[nb_])
        fd = pltpu.async_copy(dst_hbm.at[pl.ds(off, BLKE)], dstblk[nb_], sem_b[nb_])

        gd = [None] * BLK
        sd = [None] * BLK
        gd[0] = pltpu.async_copy(
            h_hbm.at[srcblk[b].at[pl.ds(0, CHUNK)]], row[0], sem_g[0])
        for kk in range(BLK):
            b2 = kk % 2
            if kk >= 1:
                sd[kk - 1].wait()
            if kk < BLK - 1:
                gd[kk + 1] = pltpu.async_copy(
                    h_hbm.at[srcblk[b].at[pl.ds((kk + 1) * CHUNK, CHUNK)]],
                    row[1 - b2], sem_g[1 - b2])
            _wcompute(blk, kk, b, b2)
            gd[kk].wait()
            _scale(b2)
            sd[kk] = pltpu.async_copy(row[b2], acc_sh.at[dstb[b2]],
                                      sem_s[b2], add=True)
        sd[BLK - 1].wait()
        fs.wait()
        fd.wait()

    pltpu.sync_copy(src_hbm.at[pl.ds(ebase, BLKE)], srcblk0)
    pltpu.sync_copy(dst_hbm.at[pl.ds(ebase, BLKE)], dstblk0)

    plsc.subcore_barrier()

    def _outer(o, _):
        blk = o * 2

        @pl.when(c == 0)
        def _():
            _block_body(hl_hbm, blk, 1)
            _block_body(hl_hbm, blk + 1, 0)

        @pl.when(c != 0)
        def _():
            _block_body(hr_hbm, blk, 1)
            _block_body(hr_hbm, blk + 1, 0)
        return 0

    lax.fori_loop(0, N_BLKS // 2, _outer, 0)

    pltpu.sync_copy(denom_v, dsh_sh.at[idxr], add=True)

    plsc.subcore_barrier()

    @pl.when(c == 0)
    def _():
        pltpu.sync_copy(acc_sh.at[pl.ds(rbase, ROWS_PER_TILE)],
                        sl_hbm.at[pl.ds(rbase, ROWS_PER_TILE)])

        @pl.when(s == NS - 1)
        def _():
            pltpu.sync_copy(acc_sh.at[pl.ds(NS * ROWS_PER_TILE, tail)],
                            sl_hbm.at[pl.ds(NS * ROWS_PER_TILE, tail)])

        @pl.when(s == 0)
        def _():
            pltpu.sync_copy(dsh_sh, den_hbm)

    @pl.when(c != 0)
    def _():
        pltpu.sync_copy(acc_sh.at[pl.ds(rbase, ROWS_PER_TILE)],
                        sr_hbm.at[pl.ds(rbase, ROWS_PER_TILE)])

        @pl.when(s == NS - 1)
        def _():
            pltpu.sync_copy(acc_sh.at[pl.ds(NS * ROWS_PER_TILE, tail)],
                            sr_hbm.at[pl.ds(NS * ROWS_PER_TILE, tail)])


def _edge_sc(src, dst, asv, adv, hl, hr):
    mesh = plsc.VectorSubcoreMesh(core_axis_name="c", subcore_axis_name="s",
                                  num_cores=NC, num_subcores=NS)
    f = pl.kernel(
        _edge_sc_body,
        out_type=(
            jax.ShapeDtypeStruct((N, 128), jnp.float32),
            jax.ShapeDtypeStruct((N, 128), jnp.float32),
            jax.ShapeDtypeStruct((DR, 128), jnp.float32),
        ),
        mesh=mesh,
        scratch_types=[
            pltpu.VMEM((N,), jnp.float32),
            pltpu.VMEM((N,), jnp.float32),
            pltpu.VMEM((BLKE,), jnp.int32),
            pltpu.VMEM((BLKE,), jnp.int32),
            pltpu.VMEM((BLKE,), jnp.int32),
            pltpu.VMEM((BLKE,), jnp.int32),
            pltpu.VMEM((CHUNK,), jnp.float32),
            pltpu.VMEM((CHUNK,), jnp.float32),
            pltpu.VMEM((CHUNK,), jnp.int32),
            pltpu.VMEM((CHUNK,), jnp.int32),
            pltpu.VMEM((CHUNK, 128), jnp.float32),
            pltpu.VMEM((CHUNK, 128), jnp.float32),
            pltpu.VMEM((DR, 128), jnp.float32),
            pltpu.VMEM((DR,), jnp.int32),
            pltpu.VMEM_SHARED((N, 128), jnp.float32),
            pltpu.VMEM_SHARED((DR, 128), jnp.float32),
            pltpu.SemaphoreType.DMA,
            pltpu.SemaphoreType.DMA,
            pltpu.SemaphoreType.DMA,
            pltpu.SemaphoreType.DMA,
            pltpu.SemaphoreType.DMA,
            pltpu.SemaphoreType.DMA,
        ],
        compiler_params=pltpu.CompilerParams(needs_layout_passes=False),
    )
    return f(src, dst, asv, adv, hl, hr)


def _edge_phase(h, asv, adv, src_pad, dst_pad):
    sl, sr, den = _edge_sc(src_pad, dst_pad, asv, adv, h[:, :128], h[:, 128:])
    seg = jnp.concatenate([sl, sr], axis=1)
    return seg / den.reshape(-1)[:N, None]


def kernel(x, edge_index, W1, a_src1, a_dst1, b1, g1, be1, W2, a_src2, a_dst2, b2, Wf, bf):
    loop = jnp.arange(N, dtype=edge_index.dtype)
    pad = jnp.zeros((E_IDX - E_TOT,), edge_index.dtype)
    src_pad = jnp.concatenate([edge_index[0], loop, pad])
    dst_pad = jnp.concatenate([edge_index[1], loop, pad])

    h1p = _mm(x, W1)
    as1 = h1p @ a_src1[0]
    ad1 = h1p @ a_dst1[0]
    u = _edge_phase(h1p, as1, ad1, src_pad, dst_pad) + b1
    m = u.mean(axis=0)
    v = u.var(axis=0)
    h1 = (u - m) / jnp.sqrt(v + 1e-5) * g1 + be1
    h1 = jax.nn.elu(h1)

    h2p = _mm(h1, W2)
    as2 = h2p @ a_src2[0]
    ad2 = h2p @ a_dst2[0]
    h2 = _edge_phase(h2p, as2, ad2, src_pad, dst_pad) + b2

    jk = jnp.maximum(h1, h2)
    return _mm(jk, Wf) + bf


# speedup vs baseline: 13.5269x; 1.0329x over previous
import jax
import jax.numpy as jnp
from jax import lax
from jax.experimental import pallas as pl
from jax.experimental.pallas import tpu as pltpu
from jax.experimental.pallas import tpu_sc as plsc

N = 10000
E = 160000
D = 256
C = 256
OUT = 128

NC = 2
NS = 16
LN = 16

ROWS_PER_TILE = 624
E_TOT = E + N
CHUNK = 64
BLK = 4
BLKE = BLK * CHUNK
N_CHUNKS = 168
N_BLKS = N_CHUNKS // BLK
EPT = N_CHUNKS * CHUNK
E_PAD = NS * EPT
E_IDX = E_PAD + BLKE
DR = 79

_BM = 1000


def _k1_body(x_ref, w_ref, a_ref, hl_ref, hr_ref, as_ref):
    h = jnp.dot(x_ref[...], w_ref[...], preferred_element_type=jnp.float32)
    hl_ref[...] = h[:, :128]
    hr_ref[...] = h[:, 128:]
    as_ref[...] = jnp.dot(h, a_ref[...], preferred_element_type=jnp.float32)


def _k1(x, W, A):
    return pl.pallas_call(
        _k1_body,
        grid=(N // _BM,),
        in_specs=[
            pl.BlockSpec((_BM, D), lambda i: (i, 0)),
            pl.BlockSpec((D, C), lambda i: (0, 0)),
            pl.BlockSpec((C, 128), lambda i: (0, 0)),
        ],
        out_specs=[
            pl.BlockSpec((_BM, 128), lambda i: (i, 0)),
            pl.BlockSpec((_BM, 128), lambda i: (i, 0)),
            pl.BlockSpec((_BM, 128), lambda i: (i, 0)),
        ],
        out_shape=[
            jax.ShapeDtypeStruct((N, 128), jnp.float32),
            jax.ShapeDtypeStruct((N, 128), jnp.float32),
            jax.ShapeDtypeStruct((N, 128), jnp.float32),
        ],
    )(x, W, A)


def _k3_body(sl_ref, sr_ref, den_ref, b_ref, g_ref, be_ref, w2_ref, a2_ref,
             h1_ref, h2l_ref, h2r_ref, as2_ref, acc_ref):
    p = pl.program_id(0)
    j = pl.program_id(1)
    u = jnp.concatenate([sl_ref[...], sr_ref[...]], axis=1)
    u = u / den_ref[...] + b_ref[...]

    @pl.when(p == 0)
    def _():
        @pl.when(j == 0)
        def _():
            acc_ref[...] = jnp.zeros_like(acc_ref)
        acc_ref[0:1, :] += jnp.sum(u, axis=0, keepdims=True)
        acc_ref[1:2, :] += jnp.sum(u * u, axis=0, keepdims=True)

    @pl.when(p == 1)
    def _():
        m = acc_ref[0:1, :] / N
        v = acc_ref[1:2, :] / N - m * m
        h1 = (u - m) * lax.rsqrt(v + 1e-5) * g_ref[...] + be_ref[...]
        h1 = jnp.where(h1 > 0, h1, jnp.exp(jnp.minimum(h1, 0.0)) - 1.0)
        h1_ref[...] = h1
        h2p = jnp.dot(h1, w2_ref[...], preferred_element_type=jnp.float32)
        h2l_ref[...] = h2p[:, :128]
        h2r_ref[...] = h2p[:, 128:]
        as2_ref[...] = jnp.dot(h2p, a2_ref[...], preferred_element_type=jnp.float32)


def _k3(sl, sr, den, b, g, be, W2, A2):
    return pl.pallas_call(
        _k3_body,
        grid=(2, N // _BM),
        in_specs=[
            pl.BlockSpec((_BM, 128), lambda p, j: (j, 0)),
            pl.BlockSpec((_BM, 128), lambda p, j: (j, 0)),
            pl.BlockSpec((_BM, 1), lambda p, j: (j, 0)),
            pl.BlockSpec((1, C), lambda p, j: (0, 0)),
            pl.BlockSpec((1, C), lambda p, j: (0, 0)),
            pl.BlockSpec((1, C), lambda p, j: (0, 0)),
            pl.BlockSpec((C, C), lambda p, j: (0, 0)),
            pl.BlockSpec((C, 128), lambda p, j: (0, 0)),
        ],
        out_specs=[
            pl.BlockSpec((_BM, C), lambda p, j: (j, 0)),
            pl.BlockSpec((_BM, 128), lambda p, j: (j, 0)),
            pl.BlockSpec((_BM, 128), lambda p, j: (j, 0)),
            pl.BlockSpec((_BM, 128), lambda p, j: (j, 0)),
        ],
        out_shape=[
            jax.ShapeDtypeStruct((N, C), jnp.float32),
            jax.ShapeDtypeStruct((N, 128), jnp.float32),
            jax.ShapeDtypeStruct((N, 128), jnp.float32),
            jax.ShapeDtypeStruct((N, 128), jnp.float32),
        ],
        scratch_shapes=[pltpu.VMEM((8, C), jnp.float32)],
    )(sl, sr, den, b, g, be, W2, A2)


def _k5_body(sl_ref, sr_ref, den_ref, b_ref, h1_ref, wf_ref, bf_ref, o_ref):
    h2 = jnp.concatenate([sl_ref[...], sr_ref[...]], axis=1)
    h2 = h2 / den_ref[...] + b_ref[...]
    jk = jnp.maximum(h1_ref[...], h2)
    o_ref[...] = jnp.dot(jk, wf_ref[...], preferred_element_type=jnp.float32) + bf_ref[...]


def _k5(sl, sr, den, b, h1, Wf, bf):
    return pl.pallas_call(
        _k5_body,
        grid=(N // _BM,),
        in_specs=[
            pl.BlockSpec((_BM, 128), lambda j: (j, 0)),
            pl.BlockSpec((_BM, 128), lambda j: (j, 0)),
            pl.BlockSpec((_BM, 1), lambda j: (j, 0)),
            pl.BlockSpec((1, C), lambda j: (0, 0)),
            pl.BlockSpec((_BM, C), lambda j: (j, 0)),
            pl.BlockSpec((C, OUT), lambda j: (0, 0)),
            pl.BlockSpec((1, OUT), lambda j: (0, 0)),
        ],
        out_specs=pl.BlockSpec((_BM, OUT), lambda j: (j, 0)),
        out_shape=jax.ShapeDtypeStruct((N, OUT), jnp.float32),
    )(sl, sr, den, b, h1, Wf, bf)


def _rot(x, sh):
    idx = (lax.iota(jnp.int32, LN) + sh) % LN
    dnums = lax.GatherDimensionNumbers(
        offset_dims=(), collapsed_slice_dims=(0,), start_index_map=(0,))
    return lax.gather(x, idx[:, None], dnums, (1,),
                      mode=lax.GatherScatterMode.PROMISE_IN_BOUNDS)


def _edge_sc_body(src_hbm, dst_hbm, asv_hbm, adv_hbm, hl_hbm, hr_hbm,
                  sl_hbm, sr_hbm, den_hbm,
                  asv_v, adv_v,
                  srcblk0, srcblk1, dstblk0, dstblk1,
                  wch0, wch1, dstb0, dstb1, row0, row1,
                  denom_v, idxr, acc_sh, dsh_sh,
                  sem_b0, sem_b1, sem_g0, sem_g1, sem_s0, sem_s1):
    srcblk = [srcblk0, srcblk1]
    dstblk = [dstblk0, dstblk1]
    wch = [wch0, wch1]
    dstb = [dstb0, dstb1]
    row = [row0, row1]
    sem_b = [sem_b0, sem_b1]
    sem_g = [sem_g0, sem_g1]
    sem_s = [sem_s0, sem_s1]

    c = lax.axis_index("c")
    s = lax.axis_index("s")
    ebase = s * EPT
    rbase = s * ROWS_PER_TILE
    iota16 = lax.iota(jnp.int32, LN)
    zf16 = jnp.zeros((LN,), jnp.float32)

    pltpu.sync_copy(asv_hbm, asv_v)
    pltpu.sync_copy(adv_hbm, adv_v)

    def _zero2d(ref, rows):
        def body(i, _):
            def inner(j, _):
                ref[i, pl.ds(j * LN, LN)] = zf16
                return 0
            lax.fori_loop(0, 128 // LN, inner, 0)
            return 0
        lax.fori_loop(0, rows, body, 0)

    _zero2d(row0, CHUNK)
    _zero2d(denom_v, DR)
    for j in range(DR // LN + 1):
        base = min(j * LN, DR - LN)
        idxr[pl.ds(base, LN)] = base + iota16

    @pl.when(s == 0)
    def _():
        pltpu.sync_copy(denom_v, dsh_sh)
    for off in range(0, ROWS_PER_TILE, CHUNK):
        nrows = min(CHUNK, ROWS_PER_TILE - off)
        pltpu.sync_copy(row0.at[pl.ds(0, nrows)],
                        acc_sh.at[pl.ds(rbase + off, nrows)])

    tail = N - NS * ROWS_PER_TILE

    @pl.when(s == NS - 1)
    def _():
        pltpu.sync_copy(row0.at[pl.ds(0, tail)],
                        acc_sh.at[pl.ds(NS * ROWS_PER_TILE, tail)])

    def _wcompute(blk, kk, b, b2):
        for j in range(CHUNK // LN):
            s16 = srcblk[b][pl.ds(kk * CHUNK + j * LN, LN)]
            d16 = dstblk[b][pl.ds(kk * CHUNK + j * LN, LN)]
            dstb[b2][pl.ds(j * LN, LN)] = d16
            a = plsc.load_gather(asv_v, [s16]) + plsc.load_gather(adv_v, [d16])
            a = jnp.maximum(a, 0.2 * a)
            w = jnp.exp(a)
            eid = ebase + (blk * BLK + kk) * CHUNK + j * LN + iota16
            w = jnp.where(eid < E_TOT, w, 0.0)
            wch[b2][pl.ds(j * LN, LN)] = w

            ds_, ws = plsc.sort_key_val(d16, w)
            csum = plsc.cumsum(ws)
            d_next = _rot(ds_, 1)
            d_prev = _rot(ds_, -1)
            c_prev = _rot(csum, -1)
            is_last = (ds_ != d_next) | (iota16 == LN - 1)
            is_start = (ds_ != d_prev) & (iota16 > 0)
            row16 = lax.shift_right_logical(ds_, 7)
            col16 = lax.bitwise_and(ds_, 127)
            plsc.addupdate_scatter(denom_v, [row16, col16], csum, mask=is_last)
            plsc.addupdate_scatter(denom_v, [row16, col16], -c_prev, mask=is_start)

    def _scale(b2):
        def body(e2, _):
            for d in range(2):
                e = e2 * 2 + d
                widx = jnp.full((LN,), e, jnp.int32)
                wspl = plsc.load_gather(wch[b2], [widx])
                for cg in range(128 // LN):
                    row[b2][e, pl.ds(cg * LN, LN)] = row[b2][e, pl.ds(cg * LN, LN)] * wspl
            return 0
        lax.fori_loop(0, CHUNK // 2, body, 0)

    def _block_body(h_hbm, blk, nb_):
        b = 1 - nb_
        off = ebase + (blk + 1) * BLKE
        fs = pltpu.async_copy(src_hbm.at[pl.ds(off, BLKE)], srcblk[nb_], sem_b[nb_])
        fd = pltpu.async_copy(dst_hbm.at[pl.ds(off, BLKE)], dstblk[nb_], sem_b[nb_])

        gd = [None] * BLK
        sd = [None] * BLK
        gd[0] = pltpu.async_copy(
            h_hbm.at[srcblk[b].at[pl.ds(0, CHUNK)]], row[0], sem_g[0])
        for kk in range(BLK):
            b2 = kk % 2
            if kk >= 1:
                sd[kk - 1].wait()
            if kk < BLK - 1:
                gd[kk + 1] = pltpu.async_copy(
                    h_hbm.at[srcblk[b].at[pl.ds((kk + 1) * CHUNK, CHUNK)]],
                    row[1 - b2], sem_g[1 - b2])
            _wcompute(blk, kk, b, b2)
            gd[kk].wait()
            _scale(b2)
            sd[kk] = pltpu.async_copy(row[b2], acc_sh.at[dstb[b2]],
                                      sem_s[b2], add=True)
        sd[BLK - 1].wait()
        fs.wait()
        fd.wait()

    pltpu.sync_copy(src_hbm.at[pl.ds(ebase, BLKE)], srcblk0)
    pltpu.sync_copy(dst_hbm.at[pl.ds(ebase, BLKE)], dstblk0)

    plsc.subcore_barrier()

    def _outer(o, _):
        blk = o * 2

        @pl.when(c == 0)
        def _():
            _block_body(hl_hbm, blk, 1)
            _block_body(hl_hbm, blk + 1, 0)

        @pl.when(c != 0)
        def _():
            _block_body(hr_hbm, blk, 1)
            _block_body(hr_hbm, blk + 1, 0)
        return 0

    lax.fori_loop(0, N_BLKS // 2, _outer, 0)

    pltpu.sync_copy(denom_v, dsh_sh.at[idxr], add=True)

    plsc.subcore_barrier()

    @pl.when(c == 0)
    def _():
        pltpu.sync_copy(acc_sh.at[pl.ds(rbase, ROWS_PER_TILE)],
                        sl_hbm.at[pl.ds(rbase, ROWS_PER_TILE)])

        @pl.when(s == NS - 1)
        def _():
            pltpu.sync_copy(acc_sh.at[pl.ds(NS * ROWS_PER_TILE, tail)],
                            sl_hbm.at[pl.ds(NS * ROWS_PER_TILE, tail)])

        @pl.when(s == 0)
        def _():
            pltpu.sync_copy(dsh_sh, den_hbm)

    @pl.when(c != 0)
    def _():
        pltpu.sync_copy(acc_sh.at[pl.ds(rbase, ROWS_PER_TILE)],
                        sr_hbm.at[pl.ds(rbase, ROWS_PER_TILE)])

        @pl.when(s == NS - 1)
        def _():
            pltpu.sync_copy(acc_sh.at[pl.ds(NS * ROWS_PER_TILE, tail)],
                            sr_hbm.at[pl.ds(NS * ROWS_PER_TILE, tail)])


def _edge_sc(src, dst, asv, adv, hl, hr):
    mesh = plsc.VectorSubcoreMesh(core_axis_name="c", subcore_axis_name="s",
                                  num_cores=NC, num_subcores=NS)
    f = pl.kernel(
        _edge_sc_body,
        out_type=(
            jax.ShapeDtypeStruct((N, 128), jnp.float32),
            jax.ShapeDtypeStruct((N, 128), jnp.float32),
            jax.ShapeDtypeStruct((DR, 128), jnp.float32),
        ),
        mesh=mesh,
        scratch_types=[
            pltpu.VMEM((N,), jnp.float32),
            pltpu.VMEM((N,), jnp.float32),
            pltpu.VMEM((BLKE,), jnp.int32),
            pltpu.VMEM((BLKE,), jnp.int32),
            pltpu.VMEM((BLKE,), jnp.int32),
            pltpu.VMEM((BLKE,), jnp.int32),
            pltpu.VMEM((CHUNK,), jnp.float32),
            pltpu.VMEM((CHUNK,), jnp.float32),
            pltpu.VMEM((CHUNK,), jnp.int32),
            pltpu.VMEM((CHUNK,), jnp.int32),
            pltpu.VMEM((CHUNK, 128), jnp.float32),
            pltpu.VMEM((CHUNK, 128), jnp.float32),
            pltpu.VMEM((DR, 128), jnp.float32),
            pltpu.VMEM((DR,), jnp.int32),
            pltpu.VMEM_SHARED((N, 128), jnp.float32),
            pltpu.VMEM_SHARED((DR, 128), jnp.float32),
            pltpu.SemaphoreType.DMA,
            pltpu.SemaphoreType.DMA,
            pltpu.SemaphoreType.DMA,
            pltpu.SemaphoreType.DMA,
            pltpu.SemaphoreType.DMA,
            pltpu.SemaphoreType.DMA,
        ],
        compiler_params=pltpu.CompilerParams(needs_layout_passes=False),
    )
    return f(src, dst, asv, adv, hl, hr)


def _apad(a_src, a_dst):
    A = jnp.zeros((C, 128), jnp.float32)
    return A.at[:, 0].set(a_src[0]).at[:, 1].set(a_dst[0])


def kernel(x, edge_index, W1, a_src1, a_dst1, b1, g1, be1, W2, a_src2, a_dst2, b2, Wf, bf):
    loop = jnp.arange(N, dtype=edge_index.dtype)
    pad = jnp.zeros((E_IDX - E_TOT,), edge_index.dtype)
    src_pad = jnp.concatenate([edge_index[0], loop, pad])
    dst_pad = jnp.concatenate([edge_index[1], loop, pad])

    h1l, h1r, asad1 = _k1(x, W1, _apad(a_src1, a_dst1))
    sl1, sr1, den1 = _edge_sc(src_pad, dst_pad, asad1[:, 0], asad1[:, 1], h1l, h1r)
    h1, h2l, h2r, asad2 = _k3(sl1, sr1, den1.reshape(-1)[:N, None],
                              b1[None, :], g1[None, :], be1[None, :],
                              W2, _apad(a_src2, a_dst2))
    sl2, sr2, den2 = _edge_sc(src_pad, dst_pad, asad2[:, 0], asad2[:, 1], h2l, h2r)
    return _k5(sl2, sr2, den2.reshape(-1)[:N, None], b2[None, :], h1, Wf, bf[None, :])
